# baseline (speedup 1.0000x reference)
import functools

import jax
import jax.numpy as jnp
from jax import lax
from jax.experimental import pallas as pl
from jax.experimental.pallas import tpu as pltpu
from jax.experimental.pallas import tpu_sc as plsc

D = 64
KH = 3
NF = 32
HOUT = D - KH + 1
NOUT = HOUT * NF
CHUNK = 128


def _build_band_weights(conv_kernel):
    k = conv_kernel[:, :, 0, :]
    w = jnp.zeros((KH, D, HOUT, NF), jnp.float32)
    ii = jnp.arange(HOUT)
    for dh in range(KH):
        w = w.at[:, ii + dh, ii, :].set(k[dh][:, None, :])
    return w.reshape(KH, D, NOUT)


def _conv_body(h_ref, r_ref, t_ref, wh_ref, wr_ref, wt_ref, b_ref, o_ref):
    acc = jnp.dot(h_ref[...], wh_ref[...], preferred_element_type=jnp.float32)
    acc = acc + jnp.dot(r_ref[...], wr_ref[...], preferred_element_type=jnp.float32)
    acc = acc + jnp.dot(t_ref[...], wt_ref[...], preferred_element_type=jnp.float32)
    o_ref[...] = acc + b_ref[...]


def _conv_tc(h_g, r_g, t_g, wh, wr, wt, bias_row, block_b):
    b = h_g.shape[0]
    grid = (b // block_b,)
    row_spec = pl.BlockSpec((block_b, D), lambda i: (i, 0))
    w_spec = pl.BlockSpec((D, NOUT), lambda i: (0, 0))
    return pl.pallas_call(
        _conv_body,
        grid=grid,
        in_specs=[row_spec, row_spec, row_spec, w_spec, w_spec, w_spec,
                  pl.BlockSpec((1, NOUT), lambda i: (0, 0))],
        out_specs=pl.BlockSpec((block_b, NOUT), lambda i: (i, 0)),
        out_shape=jax.ShapeDtypeStruct((b, NOUT), jnp.float32),
    )(h_g, r_g, t_g, wh, wr, wt, bias_row)


def _gather_sc(h_idx, r_idx, t_idx, ent_tab, rel_tab):
    nw, n_chunks, _ = h_idx.shape
    rows_w = n_chunks * CHUNK
    b = nw * rows_w
    info = plsc.get_sparse_core_info()
    nc = info.num_cores

    @functools.partial(
        pl.kernel,
        mesh=plsc.VectorSubcoreMesh(core_axis_name="c", subcore_axis_name="s"),
        compiler_params=pltpu.CompilerParams(use_tc_tiling_on_sc=False),
        out_type=(
            jax.ShapeDtypeStruct((b, D), jnp.float32),
            jax.ShapeDtypeStruct((b, D), jnp.float32),
            jax.ShapeDtypeStruct((b, D), jnp.float32),
        ),
        scratch_types=[
            pltpu.VMEM((n_chunks, CHUNK), jnp.int32),
            pltpu.VMEM((n_chunks, CHUNK), jnp.int32),
            pltpu.VMEM((n_chunks, CHUNK), jnp.int32),
            pltpu.VMEM((rows_w, D), jnp.float32),
            pltpu.VMEM((rows_w, D), jnp.float32),
            pltpu.VMEM((rows_w, D), jnp.float32),
            pltpu.SemaphoreType.DMA,
        ],
    )
    def gather_kernel(hi_hbm, ri_hbm, ti_hbm, ent_hbm, rel_hbm,
                      ho_hbm, ro_hbm, to_hbm,
                      hi_v, ri_v, ti_v, hr_v, rr_v, tr_v, sem):
        wid = lax.axis_index("s") * nc + lax.axis_index("c")
        base = wid * rows_w
        pltpu.sync_copy(hi_hbm.at[wid], hi_v)
        pltpu.sync_copy(ri_hbm.at[wid], ri_v)
        pltpu.sync_copy(ti_hbm.at[wid], ti_v)
        handles = []
        for c in range(n_chunks):
            dst = pl.ds(c * CHUNK, CHUNK)
            handles.append(pltpu.async_copy(ent_hbm.at[hi_v.at[c]], hr_v.at[dst], sem))
            handles.append(pltpu.async_copy(rel_hbm.at[ri_v.at[c]], rr_v.at[dst], sem))
            handles.append(pltpu.async_copy(ent_hbm.at[ti_v.at[c]], tr_v.at[dst], sem))
        for hnd in handles:
            hnd.wait()
        out_slice = pl.ds(base, rows_w)
        pltpu.sync_copy(hr_v, ho_hbm.at[out_slice])
        pltpu.sync_copy(rr_v, ro_hbm.at[out_slice])
        pltpu.sync_copy(tr_v, to_hbm.at[out_slice])

    return gather_kernel(h_idx, r_idx, t_idx, ent_tab, rel_tab)


def kernel(inputs, entity_embeddings, relation_embeddings, conv_kernel, conv_bias):
    b = inputs.shape[0]
    idx = inputs.astype(jnp.int32)
    info = plsc.get_sparse_core_info()
    nw = info.num_cores * info.num_subcores
    n_chunks = b // (nw * CHUNK)
    h_idx = idx[:, 0].reshape(nw, n_chunks, CHUNK)
    r_idx = idx[:, 1].reshape(nw, n_chunks, CHUNK)
    t_idx = idx[:, 2].reshape(nw, n_chunks, CHUNK)
    h_g, r_g, t_g = _gather_sc(h_idx, r_idx, t_idx,
                               entity_embeddings, relation_embeddings)
    w = _build_band_weights(conv_kernel)
    bias_row = jnp.tile(conv_bias, HOUT)[None, :]
    return _conv_tc(h_g, r_g, t_g, w[0], w[1], w[2], bias_row, 512)
